# baseline (speedup 1.0000x reference)
import jax
import jax.numpy as jnp
from jax.experimental import pallas as pl
from jax.experimental.pallas import tpu as pltpu

_NUM_EMB = 8192
_DIM = 32
_TOK_BLK = 256


def _vq_body(idx_ref, emb_ref, quant_ref, enc_ref, ratio_ref, used_ref):
    step = pl.program_id(0)
    nsteps = pl.num_programs(0)

    idx = idx_ref[0]
    emb = emb_ref[...]

    iota = jax.lax.broadcasted_iota(jnp.int32, (_TOK_BLK, _NUM_EMB), 1)
    enc = (iota == idx).astype(jnp.float32)
    enc_ref[...] = enc
    quant_ref[...] = jax.lax.dot_general(
        enc, emb, (((1,), (0,)), ((), ())),
        preferred_element_type=jnp.float32,
        precision=jax.lax.Precision.HIGHEST)

    used = jnp.max(enc, axis=0, keepdims=True)

    @pl.when(step == 0)
    def _():
        used_ref[...] = used

    @pl.when(step != 0)
    def _():
        used_ref[...] = jnp.maximum(used_ref[...], used)

    @pl.when(step == nsteps - 1)
    def _():
        ratio_ref[...] = jnp.sum(used_ref[...], keepdims=True) * (1.0 / _NUM_EMB)


def kernel(inputs, embeddings):
    x = inputs.reshape(-1, _DIM)
    n = x.shape[0]
    input_sq = jnp.sum(x ** 2, axis=1, keepdims=True)
    codebook_sq = jnp.sum(embeddings ** 2, axis=1)
    dot_product = jnp.matmul(x, embeddings.T)
    distances = input_sq + codebook_sq - 2.0 * dot_product
    encoding_indices = jnp.argmin(distances, axis=-1).astype(jnp.int32)

    nb = n // _TOK_BLK
    idx3 = encoding_indices.reshape(nb, _TOK_BLK, 1)
    quant, enc, ratio = pl.pallas_call(
        _vq_body,
        grid=(nb,),
        in_specs=[
            pl.BlockSpec((1, _TOK_BLK, 1), lambda i: (i, 0, 0)),
            pl.BlockSpec((_NUM_EMB, _DIM), lambda i: (0, 0)),
        ],
        out_specs=[
            pl.BlockSpec((_TOK_BLK, _DIM), lambda i: (i, 0)),
            pl.BlockSpec((_TOK_BLK, _NUM_EMB), lambda i: (i, 0)),
            pl.BlockSpec((1, 1), lambda i: (0, 0)),
        ],
        out_shape=[
            jax.ShapeDtypeStruct((n, _DIM), jnp.float32),
            jax.ShapeDtypeStruct((n, _NUM_EMB), jnp.float32),
            jax.ShapeDtypeStruct((1, 1), jnp.float32),
        ],
        scratch_shapes=[pltpu.VMEM((1, _NUM_EMB), jnp.float32)],
    )(idx3, embeddings)
    quantized = quant.reshape(inputs.shape)
    usage_ratio = ratio[0, 0]
    loss = jnp.zeros((1,), dtype=inputs.dtype)
    return (quantized, enc, usage_ratio, loss)


# speedup vs baseline: 1.5575x; 1.5575x over previous
import jax
import jax.numpy as jnp
from jax.experimental import pallas as pl
from jax.experimental.pallas import tpu as pltpu

_NUM_EMB = 8192
_DIM = 32
_TOK_BLK = 512


def _vq_body(idx_ref, enc_ref, ratio_ref, used_ref):
    step = pl.program_id(0)
    nsteps = pl.num_programs(0)

    idx = idx_ref[0]

    iota = jax.lax.broadcasted_iota(jnp.int32, (_TOK_BLK, _NUM_EMB), 1)
    enc = (iota == idx).astype(jnp.float32)
    enc_ref[...] = enc

    used = jnp.max(enc, axis=0, keepdims=True)

    @pl.when(step == 0)
    def _():
        used_ref[...] = used

    @pl.when(step != 0)
    def _():
        used_ref[...] = jnp.maximum(used_ref[...], used)

    @pl.when(step == nsteps - 1)
    def _():
        ratio_ref[...] = jnp.sum(used_ref[...], keepdims=True) * (1.0 / _NUM_EMB)


def kernel(inputs, embeddings):
    x = inputs.reshape(-1, _DIM)
    n = x.shape[0]
    input_sq = jnp.sum(x ** 2, axis=1, keepdims=True)
    codebook_sq = jnp.sum(embeddings ** 2, axis=1)
    dot_product = jnp.matmul(x, embeddings.T)
    distances = input_sq + codebook_sq - 2.0 * dot_product
    encoding_indices = jnp.argmin(distances, axis=-1).astype(jnp.int32)

    nb = n // _TOK_BLK
    idx3 = encoding_indices.reshape(nb, _TOK_BLK, 1)
    enc, ratio = pl.pallas_call(
        _vq_body,
        grid=(nb,),
        in_specs=[
            pl.BlockSpec((1, _TOK_BLK, 1), lambda i: (i, 0, 0)),
        ],
        out_specs=[
            pl.BlockSpec((_TOK_BLK, _NUM_EMB), lambda i: (i, 0)),
            pl.BlockSpec((1, 1), lambda i: (0, 0)),
        ],
        out_shape=[
            jax.ShapeDtypeStruct((n, _NUM_EMB), jnp.float32),
            jax.ShapeDtypeStruct((1, 1), jnp.float32),
        ],
        scratch_shapes=[pltpu.VMEM((1, _NUM_EMB), jnp.float32)],
    )(idx3)
    quantized = jnp.take(embeddings, encoding_indices, axis=0).reshape(inputs.shape)
    usage_ratio = ratio[0, 0]
    loss = jnp.zeros((1,), dtype=inputs.dtype)
    return (quantized, enc, usage_ratio, loss)


# speedup vs baseline: 1.6589x; 1.0651x over previous
import jax
import jax.numpy as jnp
from jax.experimental import pallas as pl
from jax.experimental.pallas import tpu as pltpu

_NUM_EMB = 8192
_DIM = 32
_TOK_BLK = 512


def _vq_body(idx_ref, emb_ref, quant_ref, enc_ref, ratio_ref, used_ref):
    step = pl.program_id(0)
    nsteps = pl.num_programs(0)

    idx = idx_ref[0]
    emb = emb_ref[...]

    iota = jax.lax.broadcasted_iota(jnp.int32, (_TOK_BLK, _NUM_EMB), 1)
    enc = (iota == idx).astype(jnp.float32)
    enc_ref[...] = enc
    quant_ref[...] = jax.lax.dot_general(
        enc, emb, (((1,), (0,)), ((), ())),
        preferred_element_type=jnp.float32)

    used = jnp.max(enc, axis=0, keepdims=True)

    @pl.when(step == 0)
    def _():
        used_ref[...] = used

    @pl.when(step != 0)
    def _():
        used_ref[...] = jnp.maximum(used_ref[...], used)

    @pl.when(step == nsteps - 1)
    def _():
        ratio_ref[...] = jnp.sum(used_ref[...], keepdims=True) * (1.0 / _NUM_EMB)


def kernel(inputs, embeddings):
    x = inputs.reshape(-1, _DIM)
    n = x.shape[0]
    input_sq = jnp.sum(x ** 2, axis=1, keepdims=True)
    codebook_sq = jnp.sum(embeddings ** 2, axis=1)
    dot_product = jnp.matmul(x, embeddings.T)
    distances = input_sq + codebook_sq - 2.0 * dot_product
    encoding_indices = jnp.argmin(distances, axis=-1).astype(jnp.int32)

    nb = n // _TOK_BLK
    idx3 = encoding_indices.reshape(nb, _TOK_BLK, 1)
    quant, enc, ratio = pl.pallas_call(
        _vq_body,
        grid=(nb,),
        in_specs=[
            pl.BlockSpec((1, _TOK_BLK, 1), lambda i: (i, 0, 0)),
            pl.BlockSpec((_NUM_EMB, _DIM), lambda i: (0, 0)),
        ],
        out_specs=[
            pl.BlockSpec((_TOK_BLK, _DIM), lambda i: (i, 0)),
            pl.BlockSpec((_TOK_BLK, _NUM_EMB), lambda i: (i, 0)),
            pl.BlockSpec((1, 1), lambda i: (0, 0)),
        ],
        out_shape=[
            jax.ShapeDtypeStruct((n, _DIM), jnp.float32),
            jax.ShapeDtypeStruct((n, _NUM_EMB), jnp.float32),
            jax.ShapeDtypeStruct((1, 1), jnp.float32),
        ],
        scratch_shapes=[pltpu.VMEM((1, _NUM_EMB), jnp.float32)],
    )(idx3, embeddings)
    quantized = quant.reshape(inputs.shape)
    usage_ratio = ratio[0, 0]
    loss = jnp.zeros((1,), dtype=inputs.dtype)
    return (quantized, enc, usage_ratio, loss)
